# baseline (speedup 1.0000x reference)
import jax
import jax.numpy as jnp
import numpy as np
from jax.experimental import pallas as pl
from jax.experimental.pallas import tpu as pltpu

B, D, H, R, K, NL = 4096, 384, 64, 8, 2, 64
RH = R * H
GH = RH // 2
AUX_COEF = 0.05
TILE = 2048
GRID = B // TILE

_NEG = -3.0e38


def _np_threefry2x32(k0, k1, x0, x1):
    rot = [[13, 15, 26, 6], [17, 29, 16, 24]]
    ks = [k0, k1, np.uint32(k0 ^ k1 ^ np.uint32(0x1BD11BDA))]

    def rotl(x, r):
        return (x << np.uint32(r)) | (x >> np.uint32(32 - r))

    def rounds(x0, x1, rs):
        for r in rs:
            x0 = (x0 + x1).astype(np.uint32)
            x1 = (x0 ^ rotl(x1, r)).astype(np.uint32)
        return x0, x1

    x0 = (x0 + ks[0]).astype(np.uint32)
    x1 = (x1 + ks[1]).astype(np.uint32)
    for i in range(5):
        x0, x1 = rounds(x0, x1, rot[i % 2])
        x0 = (x0 + ks[(i + 1) % 3]).astype(np.uint32)
        x1 = (x1 + ks[(i + 2) % 3] + np.uint32(i + 1)).astype(np.uint32)
    return x0, x1


def _np_uniform_key42(n):
    r0, r1 = _np_threefry2x32(np.uint32(0), np.uint32(42),
                              np.zeros(n, np.uint32),
                              np.arange(n, dtype=np.uint32))
    bits = (r0 ^ r1).astype(np.uint32)
    f = (((bits >> np.uint32(9)) | np.uint32(0x3F800000)).view(np.float32)
         - np.float32(1.0))
    return np.maximum(np.float32(0.0), f)


_RAND = _np_uniform_key42(B).reshape(B, 1)
_GI = np.arange(GH)
_G4 = (_GI[:, None] // H == _GI[None, :] // H).astype(np.float32)
_NN = np.arange(NL)
_TRI = (_NN[:, None] <= _NN[None, :]).astype(np.float32)
_F4 = (_GI[:, None] % NL == _NN[None, :]).astype(np.float32)
_ONES_COL = np.ones((NL, 1), np.float32)


def _dot(a, b):
    return jnp.dot(a, b, preferred_element_type=jnp.float32)


def _moe_kernel(x1_ref, x2_ref, le_ref, gw_ref, gb_ref, uw_ref, ub_ref,
                vw_ref, vbc_ref, g4_ref, tri_ref, f4_ref,
                ones_ref, rand_ref, sel_ref, logp_ref, aux_ref,
                ma_ref, mb_ref, uc_ref, gt_ref, accp_ref, accm_ref):
    i = pl.program_id(0)
    x1 = x1_ref[...]
    x2 = x2_ref[...]
    g4 = g4_ref[...]

    @pl.when(i == 0)
    def _():
        uc_ref[...] = uw_ref[...].T
        gt_ref[...] = gw_ref[...].T
        vc = vw_ref[...].T
        leT = le_ref[...].T
        eht = jax.lax.dot_general(vc, leT, (((0,), (0,)), ((), ())),
                                  preferred_element_type=jnp.float32)
        eht = eht + vbc_ref[...]
        eha, ehb = eht[:GH], eht[GH:]
        ehna = eha / jnp.maximum(jnp.sqrt(_dot(g4, eha * eha)), 1e-12)
        ehnb = ehb / jnp.maximum(jnp.sqrt(_dot(g4, ehb * ehb)), 1e-12)
        ma_ref[...] = jnp.concatenate([ehna] * 4, axis=1) * g4
        mb_ref[...] = jnp.concatenate([ehnb] * 4, axis=1) * g4
        accp_ref[...] = jnp.zeros_like(accp_ref)
        accm_ref[...] = jnp.zeros_like(accm_ref)

    logits = (_dot(x1, gt_ref[:D]) + _dot(x2, gt_ref[D:])
              + gb_ref[...])

    r_iota = jax.lax.broadcasted_iota(jnp.int32, logits.shape, 1)
    m1 = jnp.max(logits, axis=1, keepdims=True)
    i1 = jnp.min(jnp.where(logits == m1, r_iota, R), axis=1, keepdims=True)
    lgm = jnp.where(r_iota == i1, _NEG, logits)
    m2 = jnp.max(lgm, axis=1, keepdims=True)
    i2 = jnp.min(jnp.where(lgm == m2, r_iota, R), axis=1, keepdims=True)

    e2 = jnp.exp(m2 - m1)
    w1 = 1.0 / (1.0 + e2)
    w2 = e2 / (1.0 + e2)

    p = jnp.exp(logits - m1)
    probs = p / jnp.sum(p, axis=1, keepdims=True)
    mask = ((r_iota == i1) | (r_iota == i2)).astype(jnp.float32)
    accp_ref[...] += jnp.sum(probs, axis=0, keepdims=True)
    accm_ref[...] += jnp.sum(mask, axis=0, keepdims=True)
    aux_ref[...] = (R * AUX_COEF / (B * B)) * jnp.sum(
        accp_ref[...] * accm_ref[...], axis=1, keepdims=True)

    xh = _dot(x1, uc_ref[:D]) + _dot(x2, uc_ref[D:]) + ub_ref[...]
    xa, xb = xh[:, :GH], xh[:, GH:]
    xhna = xa / jnp.maximum(jnp.sqrt(_dot(xa * xa, g4)), 1e-12)
    xhnb = xb / jnp.maximum(jnp.sqrt(_dot(xb * xb, g4)), 1e-12)

    esa = jnp.exp(_dot(xhna, ma_ref[...]))
    esb = jnp.exp(_dot(xhnb, mb_ref[...]))
    pra = esa / _dot(esa, g4)
    prb = esb / _dot(esb, g4)

    lane4 = jax.lax.broadcasted_iota(jnp.int32, pra.shape, 1) // NL
    wa = jnp.where(lane4 == i1, w1, 0.0) + jnp.where(lane4 == i2, w2, 0.0)
    lb = lane4 + 4
    wb = jnp.where(lb == i1, w1, 0.0) + jnp.where(lb == i2, w2, 0.0)
    llm_probs = _dot(pra * wa, f4_ref[...]) + _dot(prb * wb, f4_ref[...])

    csum = _dot(llm_probs, tri_ref[...])
    rand = rand_ref[...]
    cf = _dot((csum <= rand).astype(jnp.float32), ones_ref[...])
    cnt = cf.astype(jnp.int32)
    sel = jnp.where(cnt == NL, 0, cnt)
    sel_ref[...] = sel

    n_iota = jax.lax.broadcasted_iota(jnp.int32, llm_probs.shape, 1)
    psel = _dot(jnp.where(n_iota == sel, llm_probs, 0.0), ones_ref[...])
    logp_ref[...] = jnp.log(psel)


@jax.jit
def kernel(enhanced_posts_embeddings, selected_reasoning_embeddings,
           llm_embeddings, gate_W, gate_b, U_W, U_b, V_W, V_b):
    uw = U_W.reshape(RH, 2 * D)
    ub = U_b.reshape(1, RH)
    vw = V_W.reshape(RH, D)
    vbc = V_b.reshape(RH, 1)
    gb2 = gate_b.reshape(1, R)

    cspec = lambda shape: pl.BlockSpec(shape, lambda i: (0,) * len(shape))
    sel, logp, aux = pl.pallas_call(
        _moe_kernel,
        grid=(GRID,),
        in_specs=[
            pl.BlockSpec((TILE, D), lambda i: (i, 0)),
            pl.BlockSpec((TILE, D), lambda i: (i, 0)),
            cspec((NL, D)),
            cspec((R, 2 * D)),
            cspec((1, R)),
            cspec((RH, 2 * D)),
            cspec((1, RH)),
            cspec((RH, D)),
            cspec((RH, 1)),
            cspec((GH, GH)),
            cspec((NL, NL)),
            cspec((GH, NL)),
            cspec((NL, 1)),
            pl.BlockSpec((TILE, 1), lambda i: (i, 0)),
        ],
        out_specs=[
            pl.BlockSpec((TILE, 1), lambda i: (i, 0)),
            pl.BlockSpec((TILE, 1), lambda i: (i, 0)),
            pl.BlockSpec((1, 1), lambda i: (0, 0)),
        ],
        out_shape=[
            jax.ShapeDtypeStruct((B, 1), jnp.int32),
            jax.ShapeDtypeStruct((B, 1), jnp.float32),
            jax.ShapeDtypeStruct((1, 1), jnp.float32),
        ],
        scratch_shapes=[pltpu.VMEM((GH, GH), jnp.float32),
                        pltpu.VMEM((GH, GH), jnp.float32),
                        pltpu.VMEM((2 * D, RH), jnp.float32),
                        pltpu.VMEM((2 * D, R), jnp.float32),
                        pltpu.VMEM((1, R), jnp.float32),
                        pltpu.VMEM((1, R), jnp.float32)],
    )(enhanced_posts_embeddings, selected_reasoning_embeddings,
      llm_embeddings, gate_W, gb2, uw, ub, vw, vbc,
      jnp.asarray(_G4), jnp.asarray(_TRI), jnp.asarray(_F4),
      jnp.asarray(_ONES_COL), jnp.asarray(_RAND))
    return sel[:, 0], logp, aux[0, 0]


# speedup vs baseline: 1.0224x; 1.0224x over previous
import jax
import jax.numpy as jnp
import numpy as np
from jax.experimental import pallas as pl
from jax.experimental.pallas import tpu as pltpu

B, D, H, R, K, NL = 4096, 384, 64, 8, 2, 64
RH = R * H
GH = RH // 2
AUX_COEF = 0.05
TILE = 1024
GRID = B // TILE

_NEG = -3.0e38


def _np_threefry2x32(k0, k1, x0, x1):
    rot = [[13, 15, 26, 6], [17, 29, 16, 24]]
    ks = [k0, k1, np.uint32(k0 ^ k1 ^ np.uint32(0x1BD11BDA))]

    def rotl(x, r):
        return (x << np.uint32(r)) | (x >> np.uint32(32 - r))

    def rounds(x0, x1, rs):
        for r in rs:
            x0 = (x0 + x1).astype(np.uint32)
            x1 = (x0 ^ rotl(x1, r)).astype(np.uint32)
        return x0, x1

    x0 = (x0 + ks[0]).astype(np.uint32)
    x1 = (x1 + ks[1]).astype(np.uint32)
    for i in range(5):
        x0, x1 = rounds(x0, x1, rot[i % 2])
        x0 = (x0 + ks[(i + 1) % 3]).astype(np.uint32)
        x1 = (x1 + ks[(i + 2) % 3] + np.uint32(i + 1)).astype(np.uint32)
    return x0, x1


def _np_uniform_key42(n):
    r0, r1 = _np_threefry2x32(np.uint32(0), np.uint32(42),
                              np.zeros(n, np.uint32),
                              np.arange(n, dtype=np.uint32))
    bits = (r0 ^ r1).astype(np.uint32)
    f = (((bits >> np.uint32(9)) | np.uint32(0x3F800000)).view(np.float32)
         - np.float32(1.0))
    return np.maximum(np.float32(0.0), f)


_RAND = _np_uniform_key42(B).reshape(B, 1)
_GI = np.arange(GH)
_G4 = (_GI[:, None] // H == _GI[None, :] // H).astype(np.float32)
_NN = np.arange(NL)
_TRI = (_NN[:, None] <= _NN[None, :]).astype(np.float32)
_F4 = (_GI[:, None] % NL == _NN[None, :]).astype(np.float32)
_ONES_COL = np.ones((NL, 1), np.float32)


def _dot(a, b):
    return jnp.dot(a, b, preferred_element_type=jnp.float32)


def _moe_kernel(x1_ref, x2_ref, le_ref, gw_ref, gb_ref, uw_ref, ub_ref,
                vw_ref, vbc_ref, g4_ref, tri_ref, f4_ref,
                ones_ref, rand_ref, sel_ref, logp_ref, aux_ref,
                ma_ref, mb_ref, uc_ref, gt_ref, accp_ref, accm_ref):
    i = pl.program_id(0)
    x1 = x1_ref[...]
    x2 = x2_ref[...]
    g4 = g4_ref[...]

    @pl.when(i == 0)
    def _():
        uc_ref[...] = uw_ref[...].T
        gt_ref[...] = gw_ref[...].T
        vc = vw_ref[...].T
        leT = le_ref[...].T
        eht = jax.lax.dot_general(vc, leT, (((0,), (0,)), ((), ())),
                                  preferred_element_type=jnp.float32)
        eht = eht + vbc_ref[...]
        eha, ehb = eht[:GH], eht[GH:]
        ehna = eha / jnp.maximum(jnp.sqrt(_dot(g4, eha * eha)), 1e-12)
        ehnb = ehb / jnp.maximum(jnp.sqrt(_dot(g4, ehb * ehb)), 1e-12)
        ma_ref[...] = jnp.concatenate([ehna] * 4, axis=1) * g4
        mb_ref[...] = jnp.concatenate([ehnb] * 4, axis=1) * g4
        accp_ref[...] = jnp.zeros_like(accp_ref)
        accm_ref[...] = jnp.zeros_like(accm_ref)

    logits = (_dot(x1, gt_ref[:D]) + _dot(x2, gt_ref[D:])
              + gb_ref[...])

    r_iota = jax.lax.broadcasted_iota(jnp.int32, logits.shape, 1)
    m1 = jnp.max(logits, axis=1, keepdims=True)
    i1 = jnp.min(jnp.where(logits == m1, r_iota, R), axis=1, keepdims=True)
    lgm = jnp.where(r_iota == i1, _NEG, logits)
    m2 = jnp.max(lgm, axis=1, keepdims=True)
    i2 = jnp.min(jnp.where(lgm == m2, r_iota, R), axis=1, keepdims=True)

    e2 = jnp.exp(m2 - m1)
    w1 = 1.0 / (1.0 + e2)
    w2 = e2 / (1.0 + e2)

    p = jnp.exp(logits - m1)
    probs = p / jnp.sum(p, axis=1, keepdims=True)
    mask = ((r_iota == i1) | (r_iota == i2)).astype(jnp.float32)
    accp_ref[...] += jnp.sum(probs, axis=0, keepdims=True)
    accm_ref[...] += jnp.sum(mask, axis=0, keepdims=True)
    aux_ref[...] = (R * AUX_COEF / (B * B)) * jnp.sum(
        accp_ref[...] * accm_ref[...], axis=1, keepdims=True)

    xh = _dot(x1, uc_ref[:D]) + _dot(x2, uc_ref[D:]) + ub_ref[...]
    xa, xb = xh[:, :GH], xh[:, GH:]
    xhna = xa / jnp.maximum(jnp.sqrt(_dot(xa * xa, g4)), 1e-12)
    xhnb = xb / jnp.maximum(jnp.sqrt(_dot(xb * xb, g4)), 1e-12)

    esa = jnp.exp(_dot(xhna, ma_ref[...]))
    esb = jnp.exp(_dot(xhnb, mb_ref[...]))
    pra = esa / _dot(esa, g4)
    prb = esb / _dot(esb, g4)

    lane4 = jax.lax.broadcasted_iota(jnp.int32, pra.shape, 1) // NL
    wa = jnp.where(lane4 == i1, w1, 0.0) + jnp.where(lane4 == i2, w2, 0.0)
    lb = lane4 + 4
    wb = jnp.where(lb == i1, w1, 0.0) + jnp.where(lb == i2, w2, 0.0)
    llm_probs = _dot(pra * wa, f4_ref[...]) + _dot(prb * wb, f4_ref[...])

    csum = _dot(llm_probs, tri_ref[...])
    rand = rand_ref[...]
    cf = _dot((csum <= rand).astype(jnp.float32), ones_ref[...])
    cnt = cf.astype(jnp.int32)
    sel = jnp.where(cnt == NL, 0, cnt)
    sel_ref[...] = sel

    n_iota = jax.lax.broadcasted_iota(jnp.int32, llm_probs.shape, 1)
    psel = _dot(jnp.where(n_iota == sel, llm_probs, 0.0), ones_ref[...])
    logp_ref[...] = jnp.log(psel)


@jax.jit
def kernel(enhanced_posts_embeddings, selected_reasoning_embeddings,
           llm_embeddings, gate_W, gate_b, U_W, U_b, V_W, V_b):
    uw = U_W.reshape(RH, 2 * D)
    ub = U_b.reshape(1, RH)
    vw = V_W.reshape(RH, D)
    vbc = V_b.reshape(RH, 1)
    gb2 = gate_b.reshape(1, R)

    cspec = lambda shape: pl.BlockSpec(shape, lambda i: (0,) * len(shape))
    sel, logp, aux = pl.pallas_call(
        _moe_kernel,
        grid=(GRID,),
        in_specs=[
            pl.BlockSpec((TILE, D), lambda i: (i, 0)),
            pl.BlockSpec((TILE, D), lambda i: (i, 0)),
            cspec((NL, D)),
            cspec((R, 2 * D)),
            cspec((1, R)),
            cspec((RH, 2 * D)),
            cspec((1, RH)),
            cspec((RH, D)),
            cspec((RH, 1)),
            cspec((GH, GH)),
            cspec((NL, NL)),
            cspec((GH, NL)),
            cspec((NL, 1)),
            pl.BlockSpec((TILE, 1), lambda i: (i, 0)),
        ],
        out_specs=[
            pl.BlockSpec((TILE, 1), lambda i: (i, 0)),
            pl.BlockSpec((TILE, 1), lambda i: (i, 0)),
            pl.BlockSpec((1, 1), lambda i: (0, 0)),
        ],
        out_shape=[
            jax.ShapeDtypeStruct((B, 1), jnp.int32),
            jax.ShapeDtypeStruct((B, 1), jnp.float32),
            jax.ShapeDtypeStruct((1, 1), jnp.float32),
        ],
        scratch_shapes=[pltpu.VMEM((GH, GH), jnp.float32),
                        pltpu.VMEM((GH, GH), jnp.float32),
                        pltpu.VMEM((2 * D, RH), jnp.float32),
                        pltpu.VMEM((2 * D, R), jnp.float32),
                        pltpu.VMEM((1, R), jnp.float32),
                        pltpu.VMEM((1, R), jnp.float32)],
    )(enhanced_posts_embeddings, selected_reasoning_embeddings,
      llm_embeddings, gate_W, gb2, uw, ub, vw, vbc,
      jnp.asarray(_G4), jnp.asarray(_TRI), jnp.asarray(_F4),
      jnp.asarray(_ONES_COL), jnp.asarray(_RAND))
    return sel[:, 0], logp, aux[0, 0]
